# Initial kernel scaffold; baseline (speedup 1.0000x reference)
import jax
import jax.numpy as jnp
from jax.experimental import pallas as pl


def kernel(x, edge_index, W1, b1, W2, b2, W_emb, b_emb, W_dec, b_dec, W_rep, b_rep):
    raise NotImplementedError("write your pallas kernel here")


# speedup vs baseline: 43.9360x; 43.9360x over previous
import functools

import jax
import jax.numpy as jnp
from jax import lax
from jax.experimental import pallas as pl
from jax.experimental.pallas import tpu as pltpu
from jax.experimental.pallas import tpu_sc as plsc

N = 49152
E = 786432
D = 64
NC = 2
NS = 16
CH = 128
TPT = N // NS
ROWS2D = E // CH


def _sc_mesh():
    return plsc.VectorSubcoreMesh(
        core_axis_name="c", subcore_axis_name="s",
        num_cores=NC, num_subcores=NS)


def _fill(buf, n, val):
    @pl.loop(0, n // 16)
    def _(i):
        buf[pl.ds(i * 16, 16)] = jnp.full((16,), val, jnp.float32)


def _edge_stream(src2d, dst2d, table, acc, rows, sbig, dbig,
                 isem, gsems, base_row, ngrp, gs, ones=None):
    do_gather = table is not None
    GS = gs

    def big_refs(g, half):
        out = []
        if src2d is not None:
            out.append((src2d.at[pl.ds(base_row + g * GS, GS)],
                        sbig.at[pl.ds(half * GS, GS)]))
        out.append((dst2d.at[pl.ds(base_row + g * GS, GS)],
                    dbig.at[pl.ds(half * GS, GS)]))
        return out

    def issue_big(g, half):
        for s_, d_ in big_refs(g, half):
            pltpu.async_copy(s_, d_, isem)

    def wait_big(g, half):
        for s_, d_ in big_refs(g, half):
            pltpu.make_async_copy(s_, d_, isem).wait()

    def issue_group(half, sem):
        for j in range(GS):
            m = half * GS + j
            if do_gather:
                pltpu.async_copy(table.at[sbig.at[m]], rows.at[m], sem)
            else:
                pltpu.async_copy(ones, acc.at[dbig.at[m]], sem, add=True)

    def finish_group(half, sem):
        for j in range(GS):
            m = half * GS + j
            if do_gather:
                pltpu.make_async_copy(
                    table.at[sbig.at[m]], rows.at[m], sem).wait()
                pltpu.sync_copy(rows.at[m], acc.at[dbig.at[m]], add=True)
            else:
                pltpu.make_async_copy(ones, acc.at[dbig.at[m]], sem).wait()

    issue_big(0, 0)

    @pl.loop(0, ngrp // 2)
    def _(sup):
        for gb in range(2):
            g = sup * 2 + gb
            wait_big(g, gb)
            issue_group(gb, gsems[gb])
            if gb == 0:
                pl.when(sup > 0)(lambda: finish_group(1, gsems[1]))
                issue_big(g + 1, 1)
            else:
                finish_group(0, gsems[0])
                pl.when(sup < ngrp // 2 - 1)(lambda: issue_big(g + 1, 0))

    finish_group(1, gsems[1])


@functools.lru_cache(maxsize=None)
def _make_deg_kernel():
    @functools.partial(
        pl.kernel,
        out_type=jax.ShapeDtypeStruct((NC, N), jnp.float32),
        mesh=_sc_mesh(),
        compiler_params=pltpu.CompilerParams(use_tc_tiling_on_sc=False),
        scratch_types=[
            pltpu.VMEM((2 * 12, CH), jnp.int32),
            pltpu.VMEM((CH,), jnp.float32),
            pltpu.VMEM((1024,), jnp.float32),
            pltpu.VMEM_SHARED((N,), jnp.float32),
            pltpu.SemaphoreType.DMA,
            pltpu.SemaphoreType.DMA,
            pltpu.SemaphoreType.DMA,
        ],
    )
    def deg_kernel(dst2d, out_hbm, dbig, ones, zbuf, acc,
                   isem, gsema, gsemb):
        c = lax.axis_index("c")
        s = lax.axis_index("s")
        _fill(ones, CH, 1.0)
        _fill(zbuf, 1024, 0.0)

        @pl.loop(0, TPT // 1024)
        def _(j):
            pltpu.sync_copy(zbuf, acc.at[pl.ds(s * TPT + j * 1024, 1024)])
        plsc.subcore_barrier()

        rpw = ROWS2D // (NC * NS)
        _edge_stream(None, dst2d, None, acc, None, None, dbig,
                     isem, (gsema, gsemb),
                     base_row=(s * NC + c) * rpw, ngrp=rpw // 12,
                     gs=12, ones=ones)
        plsc.subcore_barrier()

        @pl.loop(0, TPT // 1024)
        def _(j):
            r0 = s * TPT + j * 1024
            pltpu.sync_copy(acc.at[pl.ds(r0, 1024)], zbuf)
            pltpu.sync_copy(zbuf, out_hbm.at[c, pl.ds(r0, 1024)])

    return deg_kernel


@functools.lru_cache(maxsize=None)
def _make_agg1_kernel():
    @functools.partial(
        pl.kernel,
        out_type=jax.ShapeDtypeStruct((NC, N, 32), jnp.float32),
        mesh=_sc_mesh(),
        compiler_params=pltpu.CompilerParams(use_tc_tiling_on_sc=False),
        scratch_types=[
            pltpu.VMEM((2 * 2, CH), jnp.int32),
            pltpu.VMEM((2 * 2, CH), jnp.int32),
            pltpu.VMEM((2 * 2, CH, 32), jnp.float32),
            pltpu.VMEM((64, 32), jnp.float32),
            pltpu.VMEM_SHARED((N, 32), jnp.float32),
            pltpu.SemaphoreType.DMA,
            pltpu.SemaphoreType.DMA,
            pltpu.SemaphoreType.DMA,
        ],
    )
    def agg1_kernel(y0, y1, src2d, dst2d, out_hbm,
                    sbig, dbig, rows, zbuf, acc, isem, gsema, gsemb):
        c = lax.axis_index("c")
        s = lax.axis_index("s")

        @pl.loop(0, 64)
        def _(i):
            zbuf[i, pl.ds(0, 16)] = jnp.zeros((16,), jnp.float32)
            zbuf[i, pl.ds(16, 16)] = jnp.zeros((16,), jnp.float32)

        @pl.loop(0, TPT // 64)
        def _(j):
            pltpu.sync_copy(zbuf, acc.at[pl.ds(s * TPT + j * 64, 64)])
        plsc.subcore_barrier()

        rpt = ROWS2D // NS

        def run(table):
            def go():
                _edge_stream(src2d, dst2d, table, acc, rows, sbig, dbig,
                             isem, (gsema, gsemb),
                             base_row=s * rpt, ngrp=rpt // 2, gs=2)
            return go
        pl.when(c == 0)(run(y0))
        pl.when(c == 1)(run(y1))
        plsc.subcore_barrier()

        @pl.loop(0, TPT // 64)
        def _(j):
            r0 = s * TPT + j * 64
            pltpu.sync_copy(acc.at[pl.ds(r0, 64)], zbuf)
            pltpu.sync_copy(zbuf, out_hbm.at[c, pl.ds(r0, 64)])

    return agg1_kernel


@functools.lru_cache(maxsize=None)
def _make_agg2_kernel():
    @functools.partial(
        pl.kernel,
        out_type=jax.ShapeDtypeStruct((NC, N, 16), jnp.float32),
        mesh=_sc_mesh(),
        compiler_params=pltpu.CompilerParams(use_tc_tiling_on_sc=False),
        scratch_types=[
            pltpu.VMEM((2 * 8, CH), jnp.int32),
            pltpu.VMEM((2 * 8, CH), jnp.int32),
            pltpu.VMEM((2 * 8, CH, 16), jnp.float32),
            pltpu.VMEM((256, 16), jnp.float32),
            pltpu.VMEM_SHARED((N, 16), jnp.float32),
            pltpu.SemaphoreType.DMA,
            pltpu.SemaphoreType.DMA,
            pltpu.SemaphoreType.DMA,
        ],
    )
    def agg2_kernel(y2, src2d, dst2d, out_hbm,
                    sbig, dbig, rows, zbuf, acc, isem, gsema, gsemb):
        c = lax.axis_index("c")
        s = lax.axis_index("s")

        @pl.loop(0, 256)
        def _(i):
            zbuf[i, pl.ds(0, 16)] = jnp.zeros((16,), jnp.float32)

        @pl.loop(0, TPT // 256)
        def _(j):
            pltpu.sync_copy(zbuf, acc.at[pl.ds(s * TPT + j * 256, 256)])
        plsc.subcore_barrier()

        rpw = ROWS2D // (NC * NS)
        _edge_stream(src2d, dst2d, y2, acc, rows, sbig, dbig,
                     isem, (gsema, gsemb),
                     base_row=(s * NC + c) * rpw, ngrp=rpw // 8, gs=8)
        plsc.subcore_barrier()

        @pl.loop(0, TPT // 256)
        def _(j):
            r0 = s * TPT + j * 256
            pltpu.sync_copy(acc.at[pl.ds(r0, 256)], zbuf)
            pltpu.sync_copy(zbuf, out_hbm.at[c, pl.ds(r0, 256)])

    return agg2_kernel


def _gelu(v):
    return 0.5 * v * (1.0 + lax.erf(v * 0.7071067811865476))


def _pos_enc():
    return jnp.sin(
        lax.broadcasted_iota(jnp.int32, (1, D), 1).astype(jnp.float32))


def _prep_body(x_ref, dp0_ref, dp1_ref, y_ref, dinv_ref):
    deg = dp0_ref[...] + dp1_ref[...] + 1.0
    dinv = lax.rsqrt(deg)
    y = dinv * (x_ref[...] + _pos_enc())
    y_ref[0] = y[:, :32]
    y_ref[1] = y[:, 32:]
    dinv_ref[...] = dinv


def _mid_body(raw_ref, dinv_ref, x_ref, w1_ref, b1_ref, w2_ref, y2_ref):
    dinv = dinv_ref[...]
    xp = x_ref[...] + _pos_enc()
    agg = jnp.concatenate([raw_ref[0], raw_ref[1]], axis=1)
    agg = dinv * agg + (dinv * dinv) * xp
    z1 = _gelu(
        jnp.dot(agg, w1_ref[...], preferred_element_type=jnp.float32)
        + b1_ref[...])
    h2 = jnp.dot(z1, w2_ref[...], preferred_element_type=jnp.float32)
    y2_ref[...] = dinv * h2


def _f1_body(p_ref, y2_ref, dinv_ref, b2_ref, z2_ref):
    dinv = dinv_ref[...]
    agg = dinv * (p_ref[0] + p_ref[1]) + dinv * y2_ref[...]
    z2_ref[...] = _gelu(agg + b2_ref[...])


def _f2_body(r_ref, we_ref, be_ref, wd_ref, bd_ref, wr_ref, br_ref,
             enc_ref, out_ref, x1_ref):
    enc = jnp.dot(r_ref[...], we_ref[...],
                  preferred_element_type=jnp.float32) + be_ref[...]
    enc_ref[...] = enc
    x1_ref[...] = jnp.dot(enc, wr_ref[...],
                          preferred_element_type=jnp.float32) + br_ref[...]
    out_ref[...] = jnp.dot(_gelu(enc), wd_ref[...],
                           preferred_element_type=jnp.float32) + bd_ref[...]


def _full(shape):
    return pl.BlockSpec(shape, lambda i: (0,) * len(shape))


def kernel(x, edge_index, W1, b1, W2, b2, W_emb, b_emb, W_dec, b_dec,
           W_rep, b_rep):
    ei = edge_index.astype(jnp.int32)
    src2d = ei[0].reshape(ROWS2D, CH)
    dst2d = ei[1].reshape(ROWS2D, CH)

    degp = _make_deg_kernel()(dst2d)
    dp = degp.reshape(NC, N, 1)

    BR = 4096
    y, dinv = pl.pallas_call(
        _prep_body,
        grid=(N // BR,),
        in_specs=[
            pl.BlockSpec((BR, D), lambda i: (i, 0)),
            pl.BlockSpec((BR, 1), lambda i: (i, 0)),
            pl.BlockSpec((BR, 1), lambda i: (i, 0)),
        ],
        out_specs=[
            pl.BlockSpec((2, BR, 32), lambda i: (0, i, 0)),
            pl.BlockSpec((BR, 1), lambda i: (i, 0)),
        ],
        out_shape=[
            jax.ShapeDtypeStruct((2, N, 32), jnp.float32),
            jax.ShapeDtypeStruct((N, 1), jnp.float32),
        ],
    )(x, dp[0], dp[1])

    raw = _make_agg1_kernel()(y[0], y[1], src2d, dst2d)

    y2 = pl.pallas_call(
        _mid_body,
        grid=(N // BR,),
        in_specs=[
            pl.BlockSpec((2, BR, 32), lambda i: (0, i, 0)),
            pl.BlockSpec((BR, 1), lambda i: (i, 0)),
            pl.BlockSpec((BR, D), lambda i: (i, 0)),
            _full((D, 2 * D)),
            _full((1, 2 * D)),
            _full((2 * D, 16)),
        ],
        out_specs=pl.BlockSpec((BR, 16), lambda i: (i, 0)),
        out_shape=jax.ShapeDtypeStruct((N, 16), jnp.float32),
    )(raw, dinv, x, W1, b1.reshape(1, -1), W2)

    p2 = _make_agg2_kernel()(y2, src2d, dst2d)

    BRF = 8192
    z2 = pl.pallas_call(
        _f1_body,
        grid=(N // BRF,),
        in_specs=[
            pl.BlockSpec((2, BRF, 16), lambda i: (0, i, 0)),
            pl.BlockSpec((BRF, 16), lambda i: (i, 0)),
            pl.BlockSpec((BRF, 1), lambda i: (i, 0)),
            _full((1, 16)),
        ],
        out_specs=pl.BlockSpec((BRF, 16), lambda i: (i, 0)),
        out_shape=jax.ShapeDtypeStruct((N, 16), jnp.float32),
    )(p2, y2, dinv, b2.reshape(1, -1))

    r = z2.reshape(8192, 96)
    wr = jnp.pad(W_rep, ((0, 0), (0, 1)))
    br = jnp.pad(b_rep, ((0, 1),)).reshape(1, 8)
    BR2 = 2048
    enc, out, x1p = pl.pallas_call(
        _f2_body,
        grid=(8192 // BR2,),
        in_specs=[
            pl.BlockSpec((BR2, 96), lambda i: (i, 0)),
            _full((96, 96)),
            _full((1, 96)),
            _full((96, 384)),
            _full((1, 384)),
            _full((96, 8)),
            _full((1, 8)),
        ],
        out_specs=[
            pl.BlockSpec((BR2, 96), lambda i: (i, 0)),
            pl.BlockSpec((BR2, 384), lambda i: (i, 0)),
            pl.BlockSpec((BR2, 8), lambda i: (i, 0)),
        ],
        out_shape=[
            jax.ShapeDtypeStruct((8192, 96), jnp.float32),
            jax.ShapeDtypeStruct((8192, 384), jnp.float32),
            jax.ShapeDtypeStruct((8192, 8), jnp.float32),
        ],
    )(r, W_emb, b_emb.reshape(1, -1), W_dec, b_dec.reshape(1, -1), wr, br)

    return x1p[:, :7], enc, out


# speedup vs baseline: 46.4270x; 1.0567x over previous
import functools

import jax
import jax.numpy as jnp
from jax import lax
from jax.experimental import pallas as pl
from jax.experimental.pallas import tpu as pltpu
from jax.experimental.pallas import tpu_sc as plsc

N = 49152
E = 786432
D = 64
NC = 2
NS = 16
CH = 128
TPT = N // NS
ROWS2D = E // CH


def _sc_mesh():
    return plsc.VectorSubcoreMesh(
        core_axis_name="c", subcore_axis_name="s",
        num_cores=NC, num_subcores=NS)


def _fill(buf, n, val):
    @pl.loop(0, n // 16)
    def _(i):
        buf[pl.ds(i * 16, 16)] = jnp.full((16,), val, jnp.float32)


def _edge_stream(src2d, dst2d, table, acc, rows, sbig, dbig,
                 isem, gsems, base_row, ngrp, gs, ones=None):
    do_gather = table is not None
    GS = gs

    def big_refs(g, half):
        out = []
        if src2d is not None:
            out.append((src2d.at[pl.ds(base_row + g * GS, GS)],
                        sbig.at[pl.ds(half * GS, GS)]))
        out.append((dst2d.at[pl.ds(base_row + g * GS, GS)],
                    dbig.at[pl.ds(half * GS, GS)]))
        return out

    def issue_big(g, half):
        for s_, d_ in big_refs(g, half):
            pltpu.async_copy(s_, d_, isem)

    def wait_big(g, half):
        for s_, d_ in big_refs(g, half):
            pltpu.make_async_copy(s_, d_, isem).wait()

    def issue_group(half, sem):
        for j in range(GS):
            m = half * GS + j
            if do_gather:
                pltpu.async_copy(table.at[sbig.at[m]], rows.at[m], sem)
            else:
                pltpu.async_copy(ones, acc.at[dbig.at[m]], sem, add=True)

    def finish_group(half, sem):
        for j in range(GS):
            m = half * GS + j
            if do_gather:
                pltpu.make_async_copy(
                    table.at[sbig.at[m]], rows.at[m], sem).wait()
                pltpu.sync_copy(rows.at[m], acc.at[dbig.at[m]], add=True)
            else:
                pltpu.make_async_copy(ones, acc.at[dbig.at[m]], sem).wait()

    issue_big(0, 0)

    @pl.loop(0, ngrp // 2)
    def _(sup):
        for gb in range(2):
            g = sup * 2 + gb
            wait_big(g, gb)
            issue_group(gb, gsems[gb])
            if gb == 0:
                pl.when(sup > 0)(lambda: finish_group(1, gsems[1]))
                issue_big(g + 1, 1)
            else:
                finish_group(0, gsems[0])
                pl.when(sup < ngrp // 2 - 1)(lambda: issue_big(g + 1, 0))

    finish_group(1, gsems[1])


@functools.lru_cache(maxsize=None)
def _make_deg_kernel():
    @functools.partial(
        pl.kernel,
        out_type=jax.ShapeDtypeStruct((NC, N), jnp.float32),
        mesh=_sc_mesh(),
        compiler_params=pltpu.CompilerParams(use_tc_tiling_on_sc=False),
        scratch_types=[
            pltpu.VMEM((2 * 12, CH), jnp.int32),
            pltpu.VMEM((CH,), jnp.float32),
            pltpu.VMEM((1024,), jnp.float32),
            pltpu.VMEM_SHARED((N,), jnp.float32),
            pltpu.SemaphoreType.DMA,
            pltpu.SemaphoreType.DMA,
            pltpu.SemaphoreType.DMA,
        ],
    )
    def deg_kernel(dst2d, out_hbm, dbig, ones, zbuf, acc,
                   isem, gsema, gsemb):
        c = lax.axis_index("c")
        s = lax.axis_index("s")
        _fill(ones, CH, 1.0)
        _fill(zbuf, 1024, 0.0)

        @pl.loop(0, TPT // 1024)
        def _(j):
            pltpu.sync_copy(zbuf, acc.at[pl.ds(s * TPT + j * 1024, 1024)])
        plsc.subcore_barrier()

        rpw = ROWS2D // (NC * NS)
        _edge_stream(None, dst2d, None, acc, None, None, dbig,
                     isem, (gsema, gsemb),
                     base_row=(s * NC + c) * rpw, ngrp=rpw // 12,
                     gs=12, ones=ones)
        plsc.subcore_barrier()

        @pl.loop(0, TPT // 1024)
        def _(j):
            r0 = s * TPT + j * 1024
            pltpu.sync_copy(acc.at[pl.ds(r0, 1024)], zbuf)
            pltpu.sync_copy(zbuf, out_hbm.at[c, pl.ds(r0, 1024)])

    return deg_kernel


@functools.lru_cache(maxsize=None)
def _make_agg1_kernel():
    @functools.partial(
        pl.kernel,
        out_type=jax.ShapeDtypeStruct((NC, N, 32), jnp.float32),
        mesh=_sc_mesh(),
        compiler_params=pltpu.CompilerParams(use_tc_tiling_on_sc=False),
        scratch_types=[
            pltpu.VMEM((2 * 2, CH), jnp.int32),
            pltpu.VMEM((2 * 2, CH), jnp.int32),
            pltpu.VMEM((2 * 2, CH, 32), jnp.float32),
            pltpu.VMEM((64, 32), jnp.float32),
            pltpu.VMEM_SHARED((N, 32), jnp.float32),
            pltpu.SemaphoreType.DMA,
            pltpu.SemaphoreType.DMA,
            pltpu.SemaphoreType.DMA,
        ],
    )
    def agg1_kernel(y0, y1, src2d, dst2d, out_hbm,
                    sbig, dbig, rows, zbuf, acc, isem, gsema, gsemb):
        c = lax.axis_index("c")
        s = lax.axis_index("s")

        @pl.loop(0, 64)
        def _(i):
            zbuf[i, pl.ds(0, 16)] = jnp.zeros((16,), jnp.float32)
            zbuf[i, pl.ds(16, 16)] = jnp.zeros((16,), jnp.float32)

        @pl.loop(0, TPT // 64)
        def _(j):
            pltpu.sync_copy(zbuf, acc.at[pl.ds(s * TPT + j * 64, 64)])
        plsc.subcore_barrier()

        rpt = ROWS2D // NS

        def run(table):
            def go():
                _edge_stream(src2d, dst2d, table, acc, rows, sbig, dbig,
                             isem, (gsema, gsemb),
                             base_row=s * rpt, ngrp=rpt // 2, gs=2)
            return go
        pl.when(c == 0)(run(y0))
        pl.when(c == 1)(run(y1))
        plsc.subcore_barrier()

        @pl.loop(0, TPT // 64)
        def _(j):
            r0 = s * TPT + j * 64
            pltpu.sync_copy(acc.at[pl.ds(r0, 64)], zbuf)
            pltpu.sync_copy(zbuf, out_hbm.at[c, pl.ds(r0, 64)])

    return agg1_kernel


@functools.lru_cache(maxsize=None)
def _make_agg2_kernel():
    @functools.partial(
        pl.kernel,
        out_type=jax.ShapeDtypeStruct((NC, N, 16), jnp.float32),
        mesh=_sc_mesh(),
        compiler_params=pltpu.CompilerParams(use_tc_tiling_on_sc=False),
        scratch_types=[
            pltpu.VMEM((2 * 8, CH), jnp.int32),
            pltpu.VMEM((2 * 8, CH), jnp.int32),
            pltpu.VMEM((2 * 8, CH, 16), jnp.float32),
            pltpu.VMEM((256, 16), jnp.float32),
            pltpu.VMEM_SHARED((N, 16), jnp.float32),
            pltpu.SemaphoreType.DMA,
            pltpu.SemaphoreType.DMA,
            pltpu.SemaphoreType.DMA,
        ],
    )
    def agg2_kernel(y2, src2d, dst2d, out_hbm,
                    sbig, dbig, rows, zbuf, acc, isem, gsema, gsemb):
        c = lax.axis_index("c")
        s = lax.axis_index("s")

        @pl.loop(0, 256)
        def _(i):
            zbuf[i, pl.ds(0, 16)] = jnp.zeros((16,), jnp.float32)

        @pl.loop(0, TPT // 256)
        def _(j):
            pltpu.sync_copy(zbuf, acc.at[pl.ds(s * TPT + j * 256, 256)])
        plsc.subcore_barrier()

        rpw = ROWS2D // (NC * NS)
        _edge_stream(src2d, dst2d, y2, acc, rows, sbig, dbig,
                     isem, (gsema, gsemb),
                     base_row=(s * NC + c) * rpw, ngrp=rpw // 8, gs=8)
        plsc.subcore_barrier()

        @pl.loop(0, TPT // 256)
        def _(j):
            r0 = s * TPT + j * 256
            pltpu.sync_copy(acc.at[pl.ds(r0, 256)], zbuf)
            pltpu.sync_copy(zbuf, out_hbm.at[c, pl.ds(r0, 256)])

    return agg2_kernel


def _gelu(v):
    return 0.5 * v * (1.0 + lax.erf(v * 0.7071067811865476))


def _pos_enc():
    return jnp.sin(
        lax.broadcasted_iota(jnp.int32, (1, D), 1).astype(jnp.float32))


def _eprep_body(ei_ref, s_ref, d_ref):
    be = s_ref.shape[0] * CH
    s_ref[...] = ei_ref[0].reshape(s_ref.shape)
    d_ref[...] = ei_ref[1].reshape(d_ref.shape)


def _prep_body(x_ref, dp0_ref, dp1_ref, y0_ref, y1_ref, dinv_ref):
    deg = dp0_ref[...] + dp1_ref[...] + 1.0
    dinv = lax.rsqrt(deg)
    y = dinv * (x_ref[...] + _pos_enc())
    y0_ref[...] = y[:, :32]
    y1_ref[...] = y[:, 32:]
    dinv_ref[...] = dinv


def _mid_body(raw_ref, dinv_ref, x_ref, w1_ref, b1_ref, w2_ref, y2_ref):
    dinv = dinv_ref[...]
    xp = x_ref[...] + _pos_enc()
    agg = jnp.concatenate([raw_ref[0], raw_ref[1]], axis=1)
    agg = dinv * agg + (dinv * dinv) * xp
    z1 = _gelu(
        jnp.dot(agg, w1_ref[...], preferred_element_type=jnp.float32)
        + b1_ref[...])
    h2 = jnp.dot(z1, w2_ref[...], preferred_element_type=jnp.float32)
    y2_ref[...] = dinv * h2


def _f1_body(p_ref, y2_ref, dinv_ref, b2_ref, z2_ref):
    dinv = dinv_ref[...]
    agg = dinv * (p_ref[0] + p_ref[1]) + dinv * y2_ref[...]
    z2_ref[...] = _gelu(agg + b2_ref[...])


def _f2_body(r_ref, we_ref, be_ref, wd_ref, bd_ref, wr_ref, br_ref,
             enc_ref, out_ref, x1_ref):
    enc = jnp.dot(r_ref[...], we_ref[...],
                  preferred_element_type=jnp.float32) + be_ref[...]
    enc_ref[...] = enc
    x1_ref[...] = jnp.dot(enc, wr_ref[...],
                          preferred_element_type=jnp.float32) + br_ref[...]
    out_ref[...] = jnp.dot(_gelu(enc), wd_ref[...],
                           preferred_element_type=jnp.float32) + bd_ref[...]


def _full(shape):
    return pl.BlockSpec(shape, lambda i: (0,) * len(shape))


def kernel(x, edge_index, W1, b1, W2, b2, W_emb, b_emb, W_dec, b_dec,
           W_rep, b_rep):
    ei = edge_index.astype(jnp.int32)
    BE = E // 8
    src2d, dst2d = pl.pallas_call(
        _eprep_body,
        grid=(8,),
        in_specs=[pl.BlockSpec((2, BE), lambda i: (0, i))],
        out_specs=[
            pl.BlockSpec((BE // CH, CH), lambda i: (i, 0)),
            pl.BlockSpec((BE // CH, CH), lambda i: (i, 0)),
        ],
        out_shape=[
            jax.ShapeDtypeStruct((ROWS2D, CH), jnp.int32),
            jax.ShapeDtypeStruct((ROWS2D, CH), jnp.int32),
        ],
    )(ei)

    degp = _make_deg_kernel()(dst2d)
    dp = degp.reshape(NC, N, 1)

    BR = 4096
    y0, y1, dinv = pl.pallas_call(
        _prep_body,
        grid=(N // BR,),
        in_specs=[
            pl.BlockSpec((BR, D), lambda i: (i, 0)),
            pl.BlockSpec((BR, 1), lambda i: (i, 0)),
            pl.BlockSpec((BR, 1), lambda i: (i, 0)),
        ],
        out_specs=[
            pl.BlockSpec((BR, 32), lambda i: (i, 0)),
            pl.BlockSpec((BR, 32), lambda i: (i, 0)),
            pl.BlockSpec((BR, 1), lambda i: (i, 0)),
        ],
        out_shape=[
            jax.ShapeDtypeStruct((N, 32), jnp.float32),
            jax.ShapeDtypeStruct((N, 32), jnp.float32),
            jax.ShapeDtypeStruct((N, 1), jnp.float32),
        ],
    )(x, dp[0], dp[1])

    raw = _make_agg1_kernel()(y0, y1, src2d, dst2d)

    y2 = pl.pallas_call(
        _mid_body,
        grid=(N // BR,),
        in_specs=[
            pl.BlockSpec((2, BR, 32), lambda i: (0, i, 0)),
            pl.BlockSpec((BR, 1), lambda i: (i, 0)),
            pl.BlockSpec((BR, D), lambda i: (i, 0)),
            _full((D, 2 * D)),
            _full((1, 2 * D)),
            _full((2 * D, 16)),
        ],
        out_specs=pl.BlockSpec((BR, 16), lambda i: (i, 0)),
        out_shape=jax.ShapeDtypeStruct((N, 16), jnp.float32),
    )(raw, dinv, x, W1, b1.reshape(1, -1), W2)

    p2 = _make_agg2_kernel()(y2, src2d, dst2d)

    BRF = 8192
    z2 = pl.pallas_call(
        _f1_body,
        grid=(N // BRF,),
        in_specs=[
            pl.BlockSpec((2, BRF, 16), lambda i: (0, i, 0)),
            pl.BlockSpec((BRF, 16), lambda i: (i, 0)),
            pl.BlockSpec((BRF, 1), lambda i: (i, 0)),
            _full((1, 16)),
        ],
        out_specs=pl.BlockSpec((BRF, 16), lambda i: (i, 0)),
        out_shape=jax.ShapeDtypeStruct((N, 16), jnp.float32),
    )(p2, y2, dinv, b2.reshape(1, -1))

    r = z2.reshape(8192, 96)
    BR2 = 2048
    enc, out, x1p = pl.pallas_call(
        _f2_body,
        grid=(8192 // BR2,),
        in_specs=[
            pl.BlockSpec((BR2, 96), lambda i: (i, 0)),
            _full((96, 96)),
            _full((1, 96)),
            _full((96, 384)),
            _full((1, 384)),
            _full((96, 7)),
            _full((1, 7)),
        ],
        out_specs=[
            pl.BlockSpec((BR2, 96), lambda i: (i, 0)),
            pl.BlockSpec((BR2, 384), lambda i: (i, 0)),
            pl.BlockSpec((BR2, 7), lambda i: (i, 0)),
        ],
        out_shape=[
            jax.ShapeDtypeStruct((8192, 96), jnp.float32),
            jax.ShapeDtypeStruct((8192, 384), jnp.float32),
            jax.ShapeDtypeStruct((8192, 7), jnp.float32),
        ],
    )(r, W_emb, b_emb.reshape(1, -1), W_dec, b_dec.reshape(1, -1),
      W_rep, b_rep.reshape(1, -1))

    return x1p, enc, out
